# baseline (speedup 1.0000x reference)
import functools

import jax
import jax.numpy as jnp
from jax import lax
from jax.experimental import pallas as pl
from jax.experimental.pallas import tpu as pltpu
from jax.experimental.pallas import tpu_sc as plsc

END_TOKEN_VAL = 2

B = 64
BSC = 16
BTC = B - BSC
V = 100000
S = 2048
L = 16
NC = 2
NS = 16
NW = NC * NS
HSTRIDE = 49920
SEG_SZ = (25088, 24960)
SEG_OFF = (0, 25088)
NSEG = len(SEG_SZ)
VT = 32
KACC = 4
BIG = 2**30

TBLK = 25600
TNB = -(-V // TBLK)


def _greedy_sc(hid, tails):
    mesh = plsc.VectorSubcoreMesh(core_axis_name="c", subcore_axis_name="s")

    @functools.partial(
        pl.kernel,
        out_type=[
            jax.ShapeDtypeStruct((BSC, 2 * L), jnp.float32),
            jax.ShapeDtypeStruct((BSC, 2 * L), jnp.int32),
        ],
        mesh=mesh,
        compiler_params=pltpu.CompilerParams(needs_layout_passes=False),
        scratch_types=(
            [pltpu.VMEM((SEG_SZ[c],), jnp.float32) for c in range(NSEG)]
            + [
                pltpu.VMEM((VT,), jnp.float32),
                pltpu.VMEM((16,), jnp.float32),
                pltpu.VMEM((16,), jnp.int32),
            ]
            + [pltpu.SemaphoreType.DMA for _ in range(NSEG)]
        ),
    )
    def k(hid_hbm, tails_hbm, m_hbm, p_hbm, *rest):
        vbufs = rest[:NSEG]
        tailbuf, mbuf, pbuf = rest[NSEG:NSEG + 3]
        sems = rest[NSEG + 3:]
        cid = lax.axis_index("c")
        sid = lax.axis_index("s")
        wid = sid * NC + cid
        row = wid // 2
        half = wid % 2
        base = half * HSTRIDE

        lane = lax.iota(jnp.int32, L)
        neginf = jnp.full((L,), -jnp.inf, jnp.float32)
        zeros = jnp.zeros((L,), jnp.int32)

        copies = [
            pltpu.async_copy(
                hid_hbm.at[row, 0, pl.ds(base + SEG_OFF[c], SEG_SZ[c])],
                vbufs[c], sems[c])
            for c in range(NSEG)
        ]
        pltpu.sync_copy(tails_hbm.at[pl.ds(wid * VT, VT)], tailbuf)

        accs = [(neginf, zeros) for _ in range(KACC)]

        for c in range(NSEG):
            copies[c].wait()
            buf = vbufs[c]
            idxs = [base + SEG_OFF[c] + k * L + lane for k in range(KACC)]

            def body(i, carry, _buf=buf):
                st = list(carry)
                for k in range(KACC):
                    mm, pp = st[2 * k], st[2 * k + 1]
                    ix = st[2 * KACC + k]
                    v = _buf[pl.ds(i * (KACC * L) + k * L, L)]
                    upd = v > mm
                    st[2 * k] = jnp.where(upd, v, mm)
                    st[2 * k + 1] = jnp.where(upd, ix, pp)
                    st[2 * KACC + k] = ix + KACC * L
                return tuple(st)

            flat = tuple(x for a in accs for x in a) + tuple(idxs)
            flat = lax.fori_loop(0, SEG_SZ[c] // (KACC * L), body, flat,
                                 unroll=4)
            accs = [(flat[2 * k], flat[2 * k + 1]) for k in range(KACC)]

        m, posi = accs[0]
        for mm, pp in accs[1:]:
            take = (mm > m) | ((mm == m) & (pp < posi))
            m = jnp.where(take, mm, m)
            posi = jnp.where(take, pp, posi)
        for j in range(VT // L):
            v = tailbuf[pl.ds(j * L, L)]
            ix = (V - VT + j * L) + lane
            upd = v > m
            m = jnp.where(upd, v, m)
            posi = jnp.where(upd, ix, posi)
        mbuf[...] = m
        pbuf[...] = posi
        pltpu.sync_copy(mbuf, m_hbm.at[row, pl.ds(half * L, L)])
        pltpu.sync_copy(pbuf, p_hbm.at[row, pl.ds(half * L, L)])

    return k(hid, tails)


TG = 16
TNG = BTC // TG


def _tc_body(upi_ref, hid_ref, p_ref, macc_ref, iacc_ref):
    j = pl.program_id(1)

    @pl.when(j == 0)
    def _():
        macc_ref[...] = jnp.full((TG, 128), -jnp.inf, jnp.float32)
        iacc_ref[...] = jnp.zeros((TG, 128), jnp.int32)

    mac = macc_ref[...]
    iac = iacc_ref[...]
    lanei = jax.lax.broadcasted_iota(jnp.int32, (TG, 128), 1)
    for k in range(TBLK // 128):
        v = hid_ref[:, 0, k * 128:(k + 1) * 128]
        ci = lanei + (j * TBLK + k * 128)
        upd = (v > mac) & (ci < V)
        mac = jnp.where(upd, v, mac)
        iac = jnp.where(upd, ci, iac)
    macc_ref[...] = mac
    iacc_ref[...] = iac

    @pl.when(j == TNB - 1)
    def _():
        rowmax = jnp.max(mac, axis=1, keepdims=True)
        cand = jnp.where(mac == rowmax, iac, BIG)
        p = jnp.min(cand, axis=1, keepdims=True)
        p_ref[...] = jnp.broadcast_to(p, (TG, 128)).astype(jnp.int32)


def _greedy_tc(hid, upi_arr):
    grid_spec = pltpu.PrefetchScalarGridSpec(
        num_scalar_prefetch=1,
        grid=(TNG, TNB),
        in_specs=[
            pl.BlockSpec((TG, 1, TBLK), lambda g, j, upi: (g + 1, 0, j)),
        ],
        out_specs=pl.BlockSpec((TG, 128), lambda g, j, upi: (g, 0)),
        scratch_shapes=[
            pltpu.VMEM((TG, 128), jnp.float32),
            pltpu.VMEM((TG, 128), jnp.int32),
        ],
    )
    return pl.pallas_call(
        _tc_body,
        grid_spec=grid_spec,
        out_shape=jax.ShapeDtypeStruct((BTC, 128), jnp.int32),
    )(upi_arr[:1], hid)


def _assemble_body(upi_ref, outids_ref, msc_ref, psc_ref, ptc_ref, flags_ref,
                   out_ref, y_ref):
    upi = upi_ref[0]
    m = msc_ref[...]
    p = psc_ref[...]
    rowmax = jnp.max(m, axis=1, keepdims=True)
    cand = jnp.where(m == rowmax, p, BIG)
    y_sc = jnp.min(cand, axis=1, keepdims=True)
    y_raw = jnp.concatenate([y_sc, ptc_ref[:, :1]], axis=0)
    y = jnp.where(flags_ref[...] != 0, y_raw,
                  END_TOKEN_VAL).astype(jnp.int32)
    cols = jax.lax.broadcasted_iota(jnp.int32, (B, S), 1)
    out_ref[...] = jnp.where(cols == upi, y, outids_ref[...])
    y_ref[...] = jnp.broadcast_to(y, (B, 128))


def _assemble_tc(upi_arr, out_ids, m_sc, p_sc, p_tc, flags_i32):
    grid_spec = pltpu.PrefetchScalarGridSpec(
        num_scalar_prefetch=1,
        grid=(1,),
        in_specs=[
            pl.BlockSpec((B, S), lambda j, upi: (0, 0)),
            pl.BlockSpec((BSC, 2 * L), lambda j, upi: (0, 0)),
            pl.BlockSpec((BSC, 2 * L), lambda j, upi: (0, 0)),
            pl.BlockSpec((BTC, 128), lambda j, upi: (0, 0)),
            pl.BlockSpec((B, 1), lambda j, upi: (0, 0)),
        ],
        out_specs=[
            pl.BlockSpec((B, S), lambda j, upi: (0, 0)),
            pl.BlockSpec((B, 128), lambda j, upi: (0, 0)),
        ],
    )
    return pl.pallas_call(
        _assemble_body,
        grid_spec=grid_spec,
        out_shape=[
            jax.ShapeDtypeStruct((B, S), jnp.int32),
            jax.ShapeDtypeStruct((B, 128), jnp.int32),
        ],
    )(upi_arr[:1], out_ids, m_sc, p_sc, p_tc, flags_i32)


def kernel(hidden_state, update_index, out_ids, flags):
    flags_i32 = flags.astype(jnp.int32)
    upi_arr = jnp.full((8,), update_index, jnp.int32)
    tails_rows = hidden_state[:BSC, 0, V - VT:]
    tails = jnp.full((NW, VT), -jnp.inf, jnp.float32)
    tails = tails.at[1::2].set(tails_rows).reshape(NW * VT)

    m_sc, p_sc = _greedy_sc(hidden_state, tails)
    p_tc = _greedy_tc(hidden_state, upi_arr)
    out, y = _assemble_tc(upi_arr, out_ids, m_sc, p_sc, p_tc, flags_i32)
    new_flags = y[:, :1] != END_TOKEN_VAL
    return out, new_flags


# speedup vs baseline: 2.0873x; 2.0873x over previous
import functools

import jax
import jax.numpy as jnp
from jax import lax
from jax.experimental import pallas as pl
from jax.experimental.pallas import tpu as pltpu
from jax.experimental.pallas import tpu_sc as plsc

END_TOKEN_VAL = 2

B = 64
BSC = 32
BTC = B - BSC
V = 100000
S = 2048
L = 16
NC = 2
NS = 16
NW = NC * NS
_SEG_SZ = [25088, 25088, 24960, 24832]
SEGS = []
_off = 0
for _sz in _SEG_SZ:
    SEGS.append((_off, _sz))
    _off += _sz
VT = V - _off
NSEG = len(SEGS)
KACC = 4
BIG = 2**30

TBLK = 12800
TNB = -(-V // TBLK)


def _greedy_sc(hid):
    mesh = plsc.VectorSubcoreMesh(core_axis_name="c", subcore_axis_name="s")

    @functools.partial(
        pl.kernel,
        out_type=[
            jax.ShapeDtypeStruct((BSC, L), jnp.float32),
            jax.ShapeDtypeStruct((BSC, L), jnp.int32),
        ],
        mesh=mesh,
        compiler_params=pltpu.CompilerParams(needs_layout_passes=False),
        scratch_types=(
            [pltpu.VMEM((SEGS[c][1],), jnp.float32) for c in range(NSEG)]
            + [
                pltpu.VMEM((16,), jnp.float32),
                pltpu.VMEM((16,), jnp.int32),
            ]
            + [pltpu.SemaphoreType.DMA for _ in range(NSEG)]
        ),
    )
    def k(hid_hbm, m_hbm, p_hbm, *rest):
        vbufs = rest[:NSEG]
        mbuf, pbuf = rest[NSEG:NSEG + 2]
        sems = rest[NSEG + 2:]
        cid = lax.axis_index("c")
        sid = lax.axis_index("s")
        row = sid * NC + cid

        lane = lax.iota(jnp.int32, L)
        neginf = jnp.full((L,), -jnp.inf, jnp.float32)
        zeros = jnp.zeros((L,), jnp.int32)

        copies = [
            pltpu.async_copy(
                hid_hbm.at[row, 0, pl.ds(SEGS[c][0], SEGS[c][1])],
                vbufs[c], sems[c])
            for c in range(NSEG)
        ]

        accs = [(neginf, zeros) for _ in range(KACC)]

        for c in range(NSEG):
            off, sz = SEGS[c]
            copies[c].wait()
            buf = vbufs[c]
            idxs = [off + k * L + lane for k in range(KACC)]

            def body(i, carry, _buf=buf):
                st = list(carry)
                for k in range(KACC):
                    mm, pp = st[2 * k], st[2 * k + 1]
                    ix = st[2 * KACC + k]
                    v = _buf[pl.ds(i * (KACC * L) + k * L, L)]
                    upd = v > mm
                    st[2 * k] = jnp.where(upd, v, mm)
                    st[2 * k + 1] = jnp.where(upd, ix, pp)
                    st[2 * KACC + k] = ix + KACC * L
                return tuple(st)

            flat = tuple(x for a in accs for x in a) + tuple(idxs)
            flat = lax.fori_loop(0, sz // (KACC * L), body, flat, unroll=4)
            accs = [(flat[2 * k], flat[2 * k + 1]) for k in range(KACC)]

        m, posi = accs[0]
        for mm, pp in accs[1:]:
            take = (mm > m) | ((mm == m) & (pp < posi))
            m = jnp.where(take, mm, m)
            posi = jnp.where(take, pp, posi)
        mbuf[...] = m
        pbuf[...] = posi
        pltpu.sync_copy(mbuf, m_hbm.at[row])
        pltpu.sync_copy(pbuf, p_hbm.at[row])

    return k(hid)


def _tc_body(hid_ref, p_ref, macc_ref, iacc_ref):
    j = pl.program_id(0)
    x = hid_ref[...].reshape(BTC, TBLK)
    colidx = (j * TBLK
              + jax.lax.broadcasted_iota(jnp.int32, (BTC, TBLK), 1))
    x = jnp.where(colidx < V, x, -jnp.inf)

    @pl.when(j == 0)
    def _():
        macc_ref[...] = jnp.full((BTC, 128), -jnp.inf, jnp.float32)
        iacc_ref[...] = jnp.zeros((BTC, 128), jnp.int32)

    mac = macc_ref[...]
    iac = iacc_ref[...]
    for k in range(TBLK // 128):
        v = x[:, k * 128:(k + 1) * 128]
        ci = colidx[:, k * 128:(k + 1) * 128]
        upd = v > mac
        mac = jnp.where(upd, v, mac)
        iac = jnp.where(upd, ci, iac)
    macc_ref[...] = mac
    iacc_ref[...] = iac

    @pl.when(j == TNB - 1)
    def _():
        rowmax = jnp.max(mac, axis=1, keepdims=True)
        cand = jnp.where(mac == rowmax, iac, BIG)
        p = jnp.min(cand, axis=1, keepdims=True)
        p_ref[...] = jnp.broadcast_to(p, (BTC, 128)).astype(jnp.int32)


def _greedy_tc(hid):
    return pl.pallas_call(
        _tc_body,
        grid=(TNB,),
        in_specs=[
            pl.BlockSpec((BTC, 1, TBLK), lambda j: (1, 0, j)),
        ],
        out_specs=pl.BlockSpec((BTC, 128), lambda j: (0, 0)),
        scratch_shapes=[
            pltpu.VMEM((BTC, 128), jnp.float32),
            pltpu.VMEM((BTC, 128), jnp.int32),
        ],
        out_shape=jax.ShapeDtypeStruct((BTC, 128), jnp.int32),
    )(hid)


def _assemble_body(upi_ref, outids_ref, msc_ref, psc_ref, tails_ref, ptc_ref,
                   flags_ref, out_ref, y_ref):
    upi = upi_ref[0]
    tci = (V - VT) + jax.lax.broadcasted_iota(jnp.int32, (BSC, VT), 1)
    m = jnp.concatenate([msc_ref[...], tails_ref[...]], axis=1)
    p = jnp.concatenate([psc_ref[...], tci], axis=1)
    rowmax = jnp.max(m, axis=1, keepdims=True)
    cand = jnp.where(m == rowmax, p, BIG)
    y_sc = jnp.min(cand, axis=1, keepdims=True)
    y_raw = jnp.concatenate([y_sc, ptc_ref[:, :1]], axis=0)
    y = jnp.where(flags_ref[...] != 0, y_raw,
                  END_TOKEN_VAL).astype(jnp.int32)
    cols = jax.lax.broadcasted_iota(jnp.int32, (B, S), 1)
    out_ref[...] = jnp.where(cols == upi, y, outids_ref[...])
    y_ref[...] = jnp.broadcast_to(y, (B, 128))


def _assemble_tc(upi_arr, out_ids, m_sc, p_sc, tails, p_tc, flags_i32):
    grid_spec = pltpu.PrefetchScalarGridSpec(
        num_scalar_prefetch=1,
        grid=(1,),
        in_specs=[
            pl.BlockSpec((B, S), lambda j, upi: (0, 0)),
            pl.BlockSpec((BSC, L), lambda j, upi: (0, 0)),
            pl.BlockSpec((BSC, L), lambda j, upi: (0, 0)),
            pl.BlockSpec((BSC, VT), lambda j, upi: (0, 0)),
            pl.BlockSpec((BTC, 128), lambda j, upi: (0, 0)),
            pl.BlockSpec((B, 1), lambda j, upi: (0, 0)),
        ],
        out_specs=[
            pl.BlockSpec((B, S), lambda j, upi: (0, 0)),
            pl.BlockSpec((B, 128), lambda j, upi: (0, 0)),
        ],
    )
    return pl.pallas_call(
        _assemble_body,
        grid_spec=grid_spec,
        out_shape=[
            jax.ShapeDtypeStruct((B, S), jnp.int32),
            jax.ShapeDtypeStruct((B, 128), jnp.int32),
        ],
    )(upi_arr[:1], out_ids, m_sc, p_sc, tails, p_tc, flags_i32)


def kernel(hidden_state, update_index, out_ids, flags):
    flags_i32 = flags.astype(jnp.int32)
    upi_arr = jnp.full((8,), update_index, jnp.int32)
    tails = hidden_state[:BSC, 0, V - VT:]

    m_sc, p_sc = _greedy_sc(hidden_state)
    p_tc = _greedy_tc(hidden_state)
    out, y = _assemble_tc(upi_arr, out_ids, m_sc, p_sc, tails, p_tc,
                          flags_i32)
    new_flags = y[:, :1] != END_TOKEN_VAL
    return out, new_flags
